# baseline (speedup 1.0000x reference)
import functools

import jax
import jax.numpy as jnp
from jax import lax
from jax.experimental import pallas as pl
from jax.experimental.pallas import tpu as pltpu
from jax.experimental.pallas import tpu_sc as plsc

_B = 16384
_F = 16
_H = 64
_MAXR = 5.0
_MINR = 0.5
_PK = 128 // _F

_info = plsc.get_sparse_core_info()
_NC = _info.num_cores
_NS = _info.num_subcores
_NW = _NC * _NS
_CHUNK = 128
_BPW = _B // _NW
_NCHUNK = _BPW // _CHUNK
_OPW = _BPW // _PK

_mesh = plsc.VectorSubcoreMesh(core_axis_name="c", subcore_axis_name="s")


@functools.partial(
    pl.kernel,
    mesh=_mesh,
    out_type=[
        jax.ShapeDtypeStruct((_B // _PK, 128), jnp.float32),
        jax.ShapeDtypeStruct((_B // _PK, 128), jnp.float32),
    ],
    scratch_types=[
        pltpu.VMEM((_BPW,), jnp.int32),
        pltpu.VMEM((_BPW,), jnp.int32),
        pltpu.VMEM((_BPW, 128), jnp.float32),
        pltpu.VMEM((_OPW, 128), jnp.float32),
        pltpu.VMEM((_OPW, 128), jnp.float32),
        pltpu.SemaphoreType.DMA,
    ],
    compiler_params=pltpu.CompilerParams(needs_layout_passes=False),
)
def _gather2(user_h, movie_h, U_h, M_h, uo_h, mo_h,
             idx_v, hi_v, rows_v, uout_v, mout_v, sem):
    wid = lax.axis_index("s") * _NC + lax.axis_index("c")
    base = wid * _BPW

    def do_table(src_idx_h, table_h, out_v):
        pltpu.sync_copy(src_idx_h.at[pl.ds(base, _BPW)], idx_v)
        for k in range(_BPW // 16):
            sl = pl.ds(k * 16, 16)
            hi_v[sl] = idx_v[sl] >> 3
        copies = [
            pltpu.async_copy(
                table_h.at[hi_v.at[pl.ds(j * _CHUNK, _CHUNK)]],
                rows_v.at[pl.ds(j * _CHUNK, _CHUNK)],
                sem,
            )
            for j in range(_NCHUNK)
        ]
        for c in copies:
            c.wait()
        lane = lax.iota(jnp.int32, 16)
        for g in range(_BPW // 16):
            s_vec = g * 16 + lane
            lo16 = (idx_v[pl.ds(g * 16, 16)] & 7) << 4
            dst_row = s_vec >> 3
            dst_colb = (s_vec & 7) << 4
            for d in range(_F):
                val = plsc.load_gather(rows_v, [s_vec, lo16 + d])
                plsc.store_scatter(out_v, [dst_row, dst_colb + d], val)

    do_table(user_h, U_h, uout_v)
    do_table(movie_h, M_h, mout_v)
    pltpu.sync_copy(uout_v, uo_h.at[pl.ds(wid * _OPW, _OPW)])
    pltpu.sync_copy(mout_v, mo_h.at[pl.ds(wid * _OPW, _OPW)])


def _mlp_body(u_ref, m_ref, w1u_ref, w1m_ref, b1_ref, w2_ref, b2_ref, o_ref):
    h = (
        jnp.dot(u_ref[...], w1u_ref[...], preferred_element_type=jnp.float32)
        + jnp.dot(m_ref[...], w1m_ref[...], preferred_element_type=jnp.float32)
        + b1_ref[...]
    )
    h = jnp.maximum(h, 0.0)
    y = jnp.dot(h, w2_ref[...], preferred_element_type=jnp.float32) + b2_ref[...]
    sig = 1.0 / (1.0 + jnp.exp(-y))
    o_ref[...] = sig * (_MAXR - _MINR) + _MINR


_mlp = pl.pallas_call(
    _mlp_body,
    out_shape=jax.ShapeDtypeStruct((_B // _PK, _PK), jnp.float32),
)


def kernel(user, movie, U, M, W1, b1, W2, b2):
    U2 = U.reshape(-1, 128)
    M2 = M.reshape(-1, 128)
    uo, mo = _gather2(user, movie, U2, M2)
    eye = jnp.eye(_PK, dtype=jnp.float32)
    w1u = jnp.kron(eye, W1[:_F])
    w1m = jnp.kron(eye, W1[_F:])
    w2b = jnp.kron(eye, W2)
    b1b = jnp.tile(b1, (_PK,)).reshape(1, _PK * _H)
    y = _mlp(uo, mo, w1u, w1m, b1b, w2b, b2.reshape(1, 1))
    return y.reshape(_B, 1)
